# baseline (speedup 1.0000x reference)
import jax
import jax.numpy as jnp
from jax.experimental import pallas as pl
from jax.experimental.pallas import tpu as pltpu

_ALPHA = 10.0
_BLOCK_ROWS = 25000


def _softkmeans_block(x_ref, c_ref, out_ref):
    x = x_ref[...]
    c = c_ref[...]
    cx = jax.lax.dot_general(
        c, x, (((1,), (1,)), ((), ())), preferred_element_type=jnp.float32
    )
    c_sq = jnp.sum(c * c, axis=1, keepdims=True)
    t = c_sq - 2.0 * cx
    m = jnp.min(t, axis=0, keepdims=True)
    e = jnp.exp(-_ALPHA * (t - m))
    s = jnp.sum(e, axis=0, keepdims=True)
    wt = jnp.sum(e * t, axis=0, keepdims=True) / s
    xsq = jnp.sum(x * x, axis=0)
    out_ref[...] = (xsq + jnp.sum(wt) / 128.0)[None, None, :]


@jax.jit
def kernel(x, centers):
    n, d = x.shape
    k = centers.shape[0]
    steps = n // _BLOCK_ROWS
    out = pl.pallas_call(
        _softkmeans_block,
        grid=(steps,),
        in_specs=[
            pl.BlockSpec((_BLOCK_ROWS, d), lambda i: (i, 0)),
            pl.BlockSpec((k, d), lambda i: (0, 0)),
        ],
        out_specs=pl.BlockSpec((1, 1, 128), lambda i: (i, 0, 0)),
        out_shape=jax.ShapeDtypeStruct((steps, 1, 128), jnp.float32),
        compiler_params=pltpu.CompilerParams(
            dimension_semantics=("parallel",),
        ),
    )(x, centers)
    return jnp.sum(out) / n


# speedup vs baseline: 1.1061x; 1.1061x over previous
import functools

import jax
import jax.numpy as jnp
from jax.experimental import pallas as pl
from jax.experimental.pallas import tpu as pltpu

_ALPHA = 10.0
_BLOCK_ROWS = 25000


def _softkmeans_block(x_ref, c_ref, out_ref, *, inv_n):
    x = x_ref[...]
    c = c_ref[...]
    cx = jax.lax.dot_general(
        c, x, (((1,), (1,)), ((), ())), preferred_element_type=jnp.float32
    )
    c_sq = jnp.sum(c * c, axis=1, keepdims=True)
    t = c_sq - 2.0 * cx
    m = jnp.min(t, axis=0, keepdims=True)
    e = jnp.exp(-_ALPHA * (t - m))
    s = jnp.sum(e, axis=0, keepdims=True)
    wt = jnp.sum(e * t, axis=0, keepdims=True) / s
    partial = (jnp.sum(wt) + jnp.sum(x * x)) * inv_n

    @pl.when(pl.program_id(0) == 0)
    def _():
        out_ref[...] = jnp.zeros_like(out_ref)

    out_ref[...] += partial


@jax.jit
def kernel(x, centers):
    n, d = x.shape
    k = centers.shape[0]
    grid = (n // _BLOCK_ROWS,)
    out = pl.pallas_call(
        functools.partial(_softkmeans_block, inv_n=1.0 / n),
        grid=grid,
        in_specs=[
            pl.BlockSpec((_BLOCK_ROWS, d), lambda i: (i, 0)),
            pl.BlockSpec((k, d), lambda i: (0, 0)),
        ],
        out_specs=pl.BlockSpec((1, 1), lambda i: (0, 0)),
        out_shape=jax.ShapeDtypeStruct((1, 1), jnp.float32),
        compiler_params=pltpu.CompilerParams(
            dimension_semantics=("arbitrary",),
        ),
    )(x, centers)
    return out[0, 0]


# speedup vs baseline: 1.1568x; 1.0458x over previous
import functools

import jax
import jax.numpy as jnp
from jax.experimental import pallas as pl
from jax.experimental.pallas import tpu as pltpu

_ALPHA = 10.0
_BLOCK_ROWS = 10000
_N_STEPS = 5


def _stream_partial(x, c):
    cx = jax.lax.dot_general(
        c, x, (((1,), (1,)), ((), ())), preferred_element_type=jnp.float32
    )
    c_sq = jnp.sum(c * c, axis=1, keepdims=True)
    t = c_sq - 2.0 * cx
    m = jnp.min(t, axis=0, keepdims=True)
    e = jnp.exp(-_ALPHA * (t - m))
    s = jnp.sum(e, axis=0, keepdims=True)
    wt = jnp.sum(e * t, axis=0, keepdims=True) / s
    return jnp.sum(wt) + jnp.sum(x * x)


def _softkmeans_block(xa_ref, xb_ref, c_ref, out_ref, *, inv_n):
    c = c_ref[...]
    partial = (_stream_partial(xa_ref[...], c)
               + _stream_partial(xb_ref[...], c)) * inv_n

    @pl.when(pl.program_id(0) == 0)
    def _():
        out_ref[...] = jnp.zeros_like(out_ref)

    out_ref[...] += partial


@jax.jit
def kernel(x, centers):
    n, d = x.shape
    k = centers.shape[0]
    out = pl.pallas_call(
        functools.partial(_softkmeans_block, inv_n=1.0 / n),
        grid=(_N_STEPS,),
        in_specs=[
            pl.BlockSpec((_BLOCK_ROWS, d), lambda i: (i, 0)),
            pl.BlockSpec((_BLOCK_ROWS, d), lambda i: (i + _N_STEPS, 0)),
            pl.BlockSpec((k, d), lambda i: (0, 0)),
        ],
        out_specs=pl.BlockSpec((1, 1), lambda i: (0, 0)),
        out_shape=jax.ShapeDtypeStruct((1, 1), jnp.float32),
        compiler_params=pltpu.CompilerParams(
            dimension_semantics=("arbitrary",),
        ),
    )(x, x, centers)
    return out[0, 0]


# speedup vs baseline: 1.2058x; 1.0424x over previous
import functools

import jax
import jax.numpy as jnp
from jax.experimental import pallas as pl
from jax.experimental.pallas import tpu as pltpu

_ALPHA = 10.0
_BLOCK_ROWS = 5000
_N_STEPS = 5


def _stream_partial(x, c):
    cx = jax.lax.dot_general(
        c, x, (((1,), (1,)), ((), ())), preferred_element_type=jnp.float32
    )
    c_sq = jnp.sum(c * c, axis=1, keepdims=True)
    t = c_sq - 2.0 * cx
    m = jnp.min(t, axis=0, keepdims=True)
    e = jnp.exp(-_ALPHA * (t - m))
    s = jnp.sum(e, axis=0, keepdims=True)
    wt = jnp.sum(e * t, axis=0, keepdims=True) / s
    return jnp.sum(wt) + jnp.sum(x * x)


def _softkmeans_block(xa_ref, xb_ref, xc_ref, xd_ref, c_ref, out_ref, *, inv_n):
    c = c_ref[...]
    partial = (_stream_partial(xa_ref[...], c)
               + _stream_partial(xb_ref[...], c)
               + _stream_partial(xc_ref[...], c)
               + _stream_partial(xd_ref[...], c)) * inv_n

    @pl.when(pl.program_id(0) == 0)
    def _():
        out_ref[...] = jnp.zeros_like(out_ref)

    out_ref[...] += partial


@jax.jit
def kernel(x, centers):
    n, d = x.shape
    k = centers.shape[0]
    out = pl.pallas_call(
        functools.partial(_softkmeans_block, inv_n=1.0 / n),
        grid=(_N_STEPS,),
        in_specs=[
            pl.BlockSpec((_BLOCK_ROWS, d), lambda i: (i, 0)),
            pl.BlockSpec((_BLOCK_ROWS, d), lambda i: (i + _N_STEPS, 0)),
            pl.BlockSpec((_BLOCK_ROWS, d), lambda i: (i + 2 * _N_STEPS, 0)),
            pl.BlockSpec((_BLOCK_ROWS, d), lambda i: (i + 3 * _N_STEPS, 0)),
            pl.BlockSpec((k, d), lambda i: (0, 0)),
        ],
        out_specs=pl.BlockSpec((1, 1), lambda i: (0, 0)),
        out_shape=jax.ShapeDtypeStruct((1, 1), jnp.float32),
        compiler_params=pltpu.CompilerParams(
            dimension_semantics=("arbitrary",),
        ),
    )(x, x, x, x, centers)
    return out[0, 0]
